# baseline (speedup 1.0000x reference)
import functools

import jax
import jax.numpy as jnp
from jax import lax
from jax.experimental import pallas as pl
from jax.experimental.pallas import tpu as pltpu
from jax.experimental.pallas import tpu_sc as plsc

B = 1024
IB = 1024
N = 4096
HN = N // 2
K = 12
NA = 4096
PACK = 32
NWORDS = NA // PACK


_CBLK = 256
_MBLK = 512


def _wt_pair(conn, ii):
    whi = jnp.zeros((_CBLK, IB), jnp.float32)
    wlo = jnp.zeros((_CBLK, IB), jnp.float32)
    for k in range(K // 2):
        w = float(2 ** (K - 1 - k))
        whi = whi + jnp.where(conn[:, k : k + 1] == ii, w, 0.0)
    for k in range(K // 2, K):
        w = float(2 ** (K - 1 - k))
        wlo = wlo + jnp.where(conn[:, k : k + 1] == ii, w, 0.0)
    return whi.astype(jnp.bfloat16), wlo.astype(jnp.bfloat16)


def _fused_body(bits_ref, conn_lo_ref, conn_hi_ref, mem_ref,
                addr_ref, packed_ref):
    bits = bits_ref[...].astype(jnp.bfloat16)
    ii = lax.broadcasted_iota(jnp.int32, (_CBLK, IB), 1)
    nt = (((1,), (1,)), ((), ()))

    whi1, wlo1 = _wt_pair(conn_lo_ref[...], ii)
    whi2, wlo2 = _wt_pair(conn_hi_ref[...], ii)
    a_low = lax.dot_general(bits, whi1, nt, preferred_element_type=jnp.float32)
    a_low = a_low + lax.dot_general(
        bits, wlo1, nt, preferred_element_type=jnp.float32)
    a_high = lax.dot_general(bits, whi2, nt, preferred_element_type=jnp.float32)
    a_high = a_high + lax.dot_general(
        bits, wlo2, nt, preferred_element_type=jnp.float32)
    addr_ref[...] = jnp.bitwise_or(
        a_low.astype(jnp.int32), lax.shift_left(a_high.astype(jnp.int32), 16))

    m = (mem_ref[...] == 1).astype(jnp.bfloat16)
    a = lax.broadcasted_iota(jnp.int32, (NA, 2 * NWORDS), 0)
    w = lax.broadcasted_iota(jnp.int32, (NA, 2 * NWORDS), 1)
    sub = a % PACK
    cond = ((a // PACK) == (w % NWORDS)) & ((sub // 16) == (w // NWORDS))
    pm = jnp.where(cond, 1 << (sub & 15), 0).astype(jnp.bfloat16)
    nn = (((1,), (0,)), ((), ()))
    pk = lax.dot_general(m, pm, nn, preferred_element_type=jnp.float32)
    pki = pk.astype(jnp.int32)
    plo = lax.slice(pki, (0, 0), (_MBLK, NWORDS))
    phi = lax.slice(pki, (0, NWORDS), (_MBLK, 2 * NWORDS))
    packed_ref[...] = jnp.bitwise_or(plo, lax.shift_left(phi, 16))


def _tc_stage(bits_u8, connections, memory):
    return pl.pallas_call(
        _fused_body,
        grid=(N // _MBLK,),
        in_specs=[
            pl.BlockSpec((B, IB), lambda i: (0, 0)),
            pl.BlockSpec((_CBLK, K), lambda i: (i, 0)),
            pl.BlockSpec((_CBLK, K), lambda i: (i + HN // _CBLK, 0)),
            pl.BlockSpec((_MBLK, NA), lambda i: (i, 0)),
        ],
        out_specs=[
            pl.BlockSpec((B, _CBLK), lambda i: (0, i)),
            pl.BlockSpec((_MBLK, NWORDS), lambda i: (i, 0)),
        ],
        out_shape=[
            jax.ShapeDtypeStruct((B, HN), jnp.int32),
            jax.ShapeDtypeStruct((N, NWORDS), jnp.int32),
        ],
    )(bits_u8, connections, connections, memory)


_NTILES = 32
_NPT = N // _NTILES
_RUN = _NPT // 2
_CB = 256
_LANES = 16


def _lookup_body(addr_hbm, packed_hbm, out_hbm,
                 tab_v, a0, a1, o0, o1, sa0, sa1, so0, so1):
    cid = lax.axis_index("c")
    sid = lax.axis_index("s")
    wid = sid * 2 + cid
    w0 = wid * _RUN

    pltpu.sync_copy(packed_hbm.at[pl.ds(w0, _RUN), :],
                    tab_v.at[pl.ds(0, _RUN)])
    pltpu.sync_copy(packed_hbm.at[pl.ds(HN + w0, _RUN), :],
                    tab_v.at[pl.ds(_RUN, _RUN)])

    lane = lax.iota(jnp.int32, _LANES)
    nvec = [lane + g * _LANES for g in range(_NPT // _LANES)]

    abuf, obuf = (a0, a1), (o0, o1)
    asem, osem = (sa0, sa1), (so0, so1)
    nchunks = B // _CB

    def start_in(c):
        return pltpu.async_copy(
            addr_hbm.at[pl.ds(c * _CB, _CB), pl.ds(w0, _RUN)],
            abuf[c % 2], asem[c % 2])

    def start_out(c):
        ov = obuf[c % 2]
        return (
            pltpu.async_copy(
                ov.at[:, pl.ds(0, _RUN)],
                out_hbm.at[pl.ds(c * _CB, _CB), pl.ds(w0, _RUN)],
                osem[c % 2]),
            pltpu.async_copy(
                ov.at[:, pl.ds(_RUN, _RUN)],
                out_hbm.at[pl.ds(c * _CB, _CB), pl.ds(HN + w0, _RUN)],
                osem[c % 2]),
        )

    in_cps = [None] * nchunks
    out_cps = [None] * nchunks
    in_cps[0] = start_in(0)
    for c in range(nchunks):
        av, ov = abuf[c % 2], obuf[c % 2]
        if c + 1 < nchunks:
            in_cps[c + 1] = start_in(c + 1)
        in_cps[c].wait()
        if c >= 2:
            for cp in out_cps[c - 2]:
                cp.wait()

        @plsc.parallel_loop(0, _CB, unroll=4)
        def _row(r):
            for gp in range(_RUN // _LANES):
                w = av[r, pl.ds(gp * _LANES, _LANES)]
                for half in range(2):
                    if half == 0:
                        a = jnp.bitwise_and(w, 0xFFF)
                    else:
                        a = lax.shift_right_logical(w, 16)
                    g = gp + 4 * half
                    word = plsc.load_gather(
                        tab_v, [nvec[g], lax.shift_right_logical(a, 5)])
                    bit = jnp.bitwise_and(
                        lax.shift_right_logical(
                            word, jnp.bitwise_and(a, 31)), 1)
                    ov[r, pl.ds(g * _LANES, _LANES)] = bit

        out_cps[c] = start_out(c)
    for c in (nchunks - 2, nchunks - 1):
        for cp in out_cps[c]:
            cp.wait()


def _lookup(addresses, packed):
    mesh = plsc.VectorSubcoreMesh(core_axis_name="c", subcore_axis_name="s")
    f = pl.kernel(
        _lookup_body,
        out_type=jax.ShapeDtypeStruct((B, N), jnp.int32),
        mesh=mesh,
        compiler_params=pltpu.CompilerParams(
            use_tc_tiling_on_sc=False, needs_layout_passes=False
        ),
        scratch_types=[
            pltpu.VMEM((_NPT, NWORDS), jnp.int32),
            pltpu.VMEM((_CB, _RUN), jnp.int32),
            pltpu.VMEM((_CB, _RUN), jnp.int32),
            pltpu.VMEM((_CB, _NPT), jnp.int32),
            pltpu.VMEM((_CB, _NPT), jnp.int32),
            pltpu.SemaphoreType.DMA,
            pltpu.SemaphoreType.DMA,
            pltpu.SemaphoreType.DMA,
            pltpu.SemaphoreType.DMA,
        ],
    )
    return f(addresses, packed)


def kernel(input_bits, connections, memory):
    bits_u8 = input_bits.astype(jnp.uint8)
    addresses, packed = _tc_stage(bits_u8, connections, memory)
    out = _lookup(addresses, packed)
    return out.astype(jnp.bool_)


# speedup vs baseline: 1.0549x; 1.0549x over previous
import functools

import jax
import jax.numpy as jnp
from jax import lax
from jax.experimental import pallas as pl
from jax.experimental.pallas import tpu as pltpu
from jax.experimental.pallas import tpu_sc as plsc

B = 1024
IB = 1024
N = 4096
HN = N // 2
K = 12
NA = 4096
PACK = 32
NWORDS = NA // PACK


_CBLK = 256
_MBLK = 512


def _wt_pair(conn, ii):
    whi = jnp.zeros((_CBLK, IB), jnp.float32)
    wlo = jnp.zeros((_CBLK, IB), jnp.float32)
    for k in range(K // 2):
        w = float(2 ** (K - 1 - k))
        whi = whi + jnp.where(conn[:, k : k + 1] == ii, w, 0.0)
    for k in range(K // 2, K):
        w = float(2 ** (K - 1 - k))
        wlo = wlo + jnp.where(conn[:, k : k + 1] == ii, w, 0.0)
    return whi.astype(jnp.bfloat16), wlo.astype(jnp.bfloat16)


def _addr_body(bits_ref, conn_lo_ref, conn_hi_ref, addr_ref):
    bits = bits_ref[...].astype(jnp.bfloat16)
    ii = lax.broadcasted_iota(jnp.int32, (_CBLK, IB), 1)
    nt = (((1,), (1,)), ((), ()))

    whi1, wlo1 = _wt_pair(conn_lo_ref[...], ii)
    whi2, wlo2 = _wt_pair(conn_hi_ref[...], ii)
    a_low = lax.dot_general(bits, whi1, nt, preferred_element_type=jnp.float32)
    a_low = a_low + lax.dot_general(
        bits, wlo1, nt, preferred_element_type=jnp.float32)
    a_high = lax.dot_general(bits, whi2, nt, preferred_element_type=jnp.float32)
    a_high = a_high + lax.dot_general(
        bits, wlo2, nt, preferred_element_type=jnp.float32)
    addr_ref[...] = jnp.bitwise_or(
        a_low.astype(jnp.int32), lax.shift_left(a_high.astype(jnp.int32), 16))


def _addresses(bits_u8, connections):
    return pl.pallas_call(
        _addr_body,
        grid=(HN // _CBLK,),
        in_specs=[
            pl.BlockSpec((B, IB), lambda i: (0, 0)),
            pl.BlockSpec((_CBLK, K), lambda i: (i, 0)),
            pl.BlockSpec((_CBLK, K), lambda i: (i + HN // _CBLK, 0)),
        ],
        out_specs=pl.BlockSpec((B, _CBLK), lambda i: (0, i)),
        out_shape=jax.ShapeDtypeStruct((B, HN), jnp.int32),
    )(bits_u8, connections, connections)


def _pack_body(mem_ref, packed_ref):
    m = (mem_ref[...] == 1).astype(jnp.bfloat16)
    a = lax.broadcasted_iota(jnp.int32, (NA, 2 * NWORDS), 0)
    w = lax.broadcasted_iota(jnp.int32, (NA, 2 * NWORDS), 1)
    sub = a % PACK
    cond = ((a // PACK) == (w % NWORDS)) & ((sub // 16) == (w // NWORDS))
    pm = jnp.where(cond, 1 << (sub & 15), 0).astype(jnp.bfloat16)
    nn = (((1,), (0,)), ((), ()))
    pk = lax.dot_general(m, pm, nn, preferred_element_type=jnp.float32)
    pki = pk.astype(jnp.int32)
    plo = lax.slice(pki, (0, 0), (_MBLK, NWORDS))
    phi = lax.slice(pki, (0, NWORDS), (_MBLK, 2 * NWORDS))
    packed_ref[...] = jnp.bitwise_or(plo, lax.shift_left(phi, 16))


def _pack_memory(memory):
    return pl.pallas_call(
        _pack_body,
        grid=(N // _MBLK,),
        in_specs=[pl.BlockSpec((_MBLK, NA), lambda i: (i, 0))],
        out_specs=pl.BlockSpec((_MBLK, NWORDS), lambda i: (i, 0)),
        out_shape=jax.ShapeDtypeStruct((N, NWORDS), jnp.int32),
    )(memory)


_NTILES = 32
_NPT = N // _NTILES
_RUN = _NPT // 2
_CB = 256
_LANES = 16


def _lookup_body(addr_hbm, packed_hbm, out_hbm,
                 tab_v, a0, a1, o0, o1, sa0, sa1, so0, so1):
    cid = lax.axis_index("c")
    sid = lax.axis_index("s")
    wid = sid * 2 + cid
    w0 = wid * _RUN

    pltpu.sync_copy(packed_hbm.at[pl.ds(w0, _RUN), :],
                    tab_v.at[pl.ds(0, _RUN)])
    pltpu.sync_copy(packed_hbm.at[pl.ds(HN + w0, _RUN), :],
                    tab_v.at[pl.ds(_RUN, _RUN)])

    lane = lax.iota(jnp.int32, _LANES)
    nvec = [lane + g * _LANES for g in range(_NPT // _LANES)]

    abuf, obuf = (a0, a1), (o0, o1)
    asem, osem = (sa0, sa1), (so0, so1)
    nchunks = B // _CB

    def start_in(c):
        return pltpu.async_copy(
            addr_hbm.at[pl.ds(c * _CB, _CB), pl.ds(w0, _RUN)],
            abuf[c % 2], asem[c % 2])

    def start_out(c):
        ov = obuf[c % 2]
        return (
            pltpu.async_copy(
                ov.at[:, pl.ds(0, _RUN)],
                out_hbm.at[pl.ds(c * _CB, _CB), pl.ds(w0, _RUN)],
                osem[c % 2]),
            pltpu.async_copy(
                ov.at[:, pl.ds(_RUN, _RUN)],
                out_hbm.at[pl.ds(c * _CB, _CB), pl.ds(HN + w0, _RUN)],
                osem[c % 2]),
        )

    in_cps = [None] * nchunks
    out_cps = [None] * nchunks
    in_cps[0] = start_in(0)
    for c in range(nchunks):
        av, ov = abuf[c % 2], obuf[c % 2]
        if c + 1 < nchunks:
            in_cps[c + 1] = start_in(c + 1)
        in_cps[c].wait()
        if c >= 2:
            for cp in out_cps[c - 2]:
                cp.wait()

        @plsc.parallel_loop(0, _CB, unroll=4)
        def _row(r):
            for gp in range(_RUN // _LANES):
                w = av[r, pl.ds(gp * _LANES, _LANES)]
                for half in range(2):
                    if half == 0:
                        a = jnp.bitwise_and(w, 0xFFF)
                    else:
                        a = lax.shift_right_logical(w, 16)
                    g = gp + 4 * half
                    word = plsc.load_gather(
                        tab_v, [nvec[g], lax.shift_right_logical(a, 5)])
                    bit = jnp.bitwise_and(
                        lax.shift_right_logical(
                            word, jnp.bitwise_and(a, 31)), 1)
                    ov[r, pl.ds(g * _LANES, _LANES)] = bit

        out_cps[c] = start_out(c)
    for c in (nchunks - 2, nchunks - 1):
        for cp in out_cps[c]:
            cp.wait()


def _lookup(addresses, packed):
    mesh = plsc.VectorSubcoreMesh(core_axis_name="c", subcore_axis_name="s")
    f = pl.kernel(
        _lookup_body,
        out_type=jax.ShapeDtypeStruct((B, N), jnp.int32),
        mesh=mesh,
        compiler_params=pltpu.CompilerParams(
            use_tc_tiling_on_sc=False, needs_layout_passes=False
        ),
        scratch_types=[
            pltpu.VMEM((_NPT, NWORDS), jnp.int32),
            pltpu.VMEM((_CB, _RUN), jnp.int32),
            pltpu.VMEM((_CB, _RUN), jnp.int32),
            pltpu.VMEM((_CB, _NPT), jnp.int32),
            pltpu.VMEM((_CB, _NPT), jnp.int32),
            pltpu.SemaphoreType.DMA,
            pltpu.SemaphoreType.DMA,
            pltpu.SemaphoreType.DMA,
            pltpu.SemaphoreType.DMA,
        ],
    )
    return f(addresses, packed)


def kernel(input_bits, connections, memory):
    bits_u8 = input_bits.astype(jnp.uint8)
    addresses = _addresses(bits_u8, connections)
    packed = _pack_memory(memory)
    out = _lookup(addresses, packed)
    return out.astype(jnp.bool_)


# speedup vs baseline: 1.1243x; 1.0657x over previous
import functools

import jax
import jax.numpy as jnp
from jax import lax
from jax.experimental import pallas as pl
from jax.experimental.pallas import tpu as pltpu
from jax.experimental.pallas import tpu_sc as plsc

B = 1024
IB = 1024
N = 4096
HN = N // 2
K = 12
NA = 4096
PACK = 32
NWORDS = NA // PACK


_CBLK = 256
_MBLK = 512


def _wt_quad(conn_refs, ii):
    whi_parts, wlo_parts = [], []
    for cr in conn_refs:
        conn = cr[...]
        whi = jnp.zeros((64, IB), jnp.float32)
        wlo = jnp.zeros((64, IB), jnp.float32)
        for k in range(K // 2):
            w = float(2 ** (K - 1 - k))
            whi = whi + jnp.where(conn[:, k : k + 1] == ii, w, 0.0)
        for k in range(K // 2, K):
            w = float(2 ** (K - 1 - k))
            wlo = wlo + jnp.where(conn[:, k : k + 1] == ii, w, 0.0)
        whi_parts.append(whi)
        wlo_parts.append(wlo)
    return (jnp.concatenate(whi_parts, axis=0).astype(jnp.bfloat16),
            jnp.concatenate(wlo_parts, axis=0).astype(jnp.bfloat16))


def _addr_body(bits_ref, c0, c1, c2, c3, c4, c5, c6, c7, addr_ref):
    bits = bits_ref[...].astype(jnp.bfloat16)
    ii = lax.broadcasted_iota(jnp.int32, (64, IB), 1)
    nt = (((1,), (1,)), ((), ()))

    whi_l, wlo_l = _wt_quad([c0, c2, c4, c6], ii)
    whi_h, wlo_h = _wt_quad([c1, c3, c5, c7], ii)
    a_low = lax.dot_general(
        bits, whi_l, nt, preferred_element_type=jnp.float32)
    a_low = a_low + lax.dot_general(
        bits, wlo_l, nt, preferred_element_type=jnp.float32)
    a_high = lax.dot_general(
        bits, whi_h, nt, preferred_element_type=jnp.float32)
    a_high = a_high + lax.dot_general(
        bits, wlo_h, nt, preferred_element_type=jnp.float32)
    addr_ref[...] = jnp.bitwise_or(
        a_low.astype(jnp.int32), lax.shift_left(a_high.astype(jnp.int32), 16))


def _addresses(bits_u8, connections):
    return pl.pallas_call(
        _addr_body,
        grid=(HN // _CBLK,),
        in_specs=[pl.BlockSpec((B, IB), lambda i: (0, 0))] + [
            pl.BlockSpec((64, K), lambda i, s=sub, h=half: (8 * i + 2 * s + h, 0))
            for sub in range(4) for half in range(2)
        ],
        out_specs=pl.BlockSpec((B, _CBLK), lambda i: (0, i)),
        out_shape=jax.ShapeDtypeStruct((B, HN), jnp.int32),
    )(bits_u8, *([connections] * 8))


def _pack_body(mem_ref, packed_ref):
    m = (mem_ref[...] == 1).astype(jnp.bfloat16)
    a = lax.broadcasted_iota(jnp.int32, (NA, 2 * NWORDS), 0)
    w = lax.broadcasted_iota(jnp.int32, (NA, 2 * NWORDS), 1)
    sub = a % PACK
    cond = ((a // PACK) == (w % NWORDS)) & ((sub // 16) == (w // NWORDS))
    pm = jnp.where(cond, 1 << (sub & 15), 0).astype(jnp.bfloat16)
    nn = (((1,), (0,)), ((), ()))
    pk = lax.dot_general(m, pm, nn, preferred_element_type=jnp.float32)
    pki = pk.astype(jnp.int32)
    plo = lax.slice(pki, (0, 0), (_MBLK, NWORDS))
    phi = lax.slice(pki, (0, NWORDS), (_MBLK, 2 * NWORDS))
    packed_ref[...] = jnp.bitwise_or(plo, lax.shift_left(phi, 16))


def _pack_memory(memory):
    return pl.pallas_call(
        _pack_body,
        grid=(N // _MBLK,),
        in_specs=[pl.BlockSpec((_MBLK, NA), lambda i: (i, 0))],
        out_specs=pl.BlockSpec((_MBLK, NWORDS), lambda i: (i, 0)),
        out_shape=jax.ShapeDtypeStruct((N, NWORDS), jnp.int32),
    )(memory)


_NTILES = 32
_NPT = N // _NTILES
_RUN = _NPT // 2
_CB = 256
_LANES = 16


def _lookup_body(addr_hbm, packed_hbm, out_hbm,
                 tab_v, a0, a1, o0, o1, sa0, sa1, so0, so1):
    cid = lax.axis_index("c")
    sid = lax.axis_index("s")
    wid = sid * 2 + cid
    n0 = wid * _NPT
    w0 = wid * _RUN

    pltpu.sync_copy(packed_hbm.at[pl.ds(n0, _NPT), :], tab_v)

    lane = lax.iota(jnp.int32, _LANES)
    nvec = [lane + g * _LANES for g in range(_NPT // _LANES)]

    abuf, obuf = (a0, a1), (o0, o1)
    asem, osem = (sa0, sa1), (so0, so1)
    nchunks = B // _CB

    def start_in(c):
        return pltpu.async_copy(
            addr_hbm.at[pl.ds(c * _CB, _CB), pl.ds(w0, _RUN)],
            abuf[c % 2], asem[c % 2])

    def start_out(c):
        return pltpu.async_copy(
            obuf[c % 2],
            out_hbm.at[pl.ds(c * _CB, _CB), pl.ds(n0, _NPT)], osem[c % 2])

    in_cps = [None] * nchunks
    out_cps = [None] * nchunks
    in_cps[0] = start_in(0)
    for c in range(nchunks):
        av, ov = abuf[c % 2], obuf[c % 2]
        if c + 1 < nchunks:
            in_cps[c + 1] = start_in(c + 1)
        in_cps[c].wait()
        if c >= 2:
            out_cps[c - 2].wait()

        @plsc.parallel_loop(0, _CB, unroll=4)
        def _row(r):
            for gp in range(_RUN // _LANES):
                w = av[r, pl.ds(gp * _LANES, _LANES)]
                for half in range(2):
                    if half == 0:
                        a = jnp.bitwise_and(w, 0xFFF)
                    else:
                        a = lax.shift_right_logical(w, 16)
                    g = gp + 4 * half
                    word = plsc.load_gather(
                        tab_v, [nvec[g], lax.shift_right_logical(a, 5)])
                    bit = jnp.bitwise_and(
                        lax.shift_right_logical(
                            word, jnp.bitwise_and(a, 31)), 1)
                    ov[r, pl.ds(g * _LANES, _LANES)] = bit

        out_cps[c] = start_out(c)
    out_cps[-2].wait()
    out_cps[-1].wait()


def _lookup(addresses, packed):
    mesh = plsc.VectorSubcoreMesh(core_axis_name="c", subcore_axis_name="s")
    f = pl.kernel(
        _lookup_body,
        out_type=jax.ShapeDtypeStruct((B, N), jnp.int32),
        mesh=mesh,
        compiler_params=pltpu.CompilerParams(
            use_tc_tiling_on_sc=False, needs_layout_passes=False
        ),
        scratch_types=[
            pltpu.VMEM((_NPT, NWORDS), jnp.int32),
            pltpu.VMEM((_CB, _RUN), jnp.int32),
            pltpu.VMEM((_CB, _RUN), jnp.int32),
            pltpu.VMEM((_CB, _NPT), jnp.int32),
            pltpu.VMEM((_CB, _NPT), jnp.int32),
            pltpu.SemaphoreType.DMA,
            pltpu.SemaphoreType.DMA,
            pltpu.SemaphoreType.DMA,
            pltpu.SemaphoreType.DMA,
        ],
    )
    return f(addresses, packed)


def kernel(input_bits, connections, memory):
    bits_u8 = input_bits.astype(jnp.uint8)
    addresses = _addresses(bits_u8, connections)
    packed = _pack_memory(memory)
    out = _lookup(addresses, packed)
    return out.astype(jnp.bool_)


# speedup vs baseline: 1.1278x; 1.0032x over previous
import functools

import jax
import jax.numpy as jnp
from jax import lax
from jax.experimental import pallas as pl
from jax.experimental.pallas import tpu as pltpu
from jax.experimental.pallas import tpu_sc as plsc

B = 1024
IB = 1024
N = 4096
HN = N // 2
K = 12
NA = 4096
PACK = 32
NWORDS = NA // PACK


_CBLK = 256
_MBLK = 512


def _wt_quad(conn_refs, ii):
    whi_parts, wlo_parts = [], []
    for cr in conn_refs:
        conn = cr[...]
        whi = jnp.zeros((64, IB), jnp.float32)
        wlo = jnp.zeros((64, IB), jnp.float32)
        for k in range(K // 2):
            w = float(2 ** (K - 1 - k))
            whi = whi + jnp.where(conn[:, k : k + 1] == ii, w, 0.0)
        for k in range(K // 2, K):
            w = float(2 ** (K - 1 - k))
            wlo = wlo + jnp.where(conn[:, k : k + 1] == ii, w, 0.0)
        whi_parts.append(whi)
        wlo_parts.append(wlo)
    return (jnp.concatenate(whi_parts, axis=0).astype(jnp.bfloat16),
            jnp.concatenate(wlo_parts, axis=0).astype(jnp.bfloat16))


def _addr_body(bits_ref, c0, c1, c2, c3, c4, c5, c6, c7, addr_ref):
    bits = bits_ref[...].astype(jnp.bfloat16)
    ii = lax.broadcasted_iota(jnp.int32, (64, IB), 1)
    nt = (((1,), (1,)), ((), ()))

    whi_l, wlo_l = _wt_quad([c0, c2, c4, c6], ii)
    whi_h, wlo_h = _wt_quad([c1, c3, c5, c7], ii)
    a_low = lax.dot_general(
        bits, whi_l, nt, preferred_element_type=jnp.float32)
    a_low = a_low + lax.dot_general(
        bits, wlo_l, nt, preferred_element_type=jnp.float32)
    a_high = lax.dot_general(
        bits, whi_h, nt, preferred_element_type=jnp.float32)
    a_high = a_high + lax.dot_general(
        bits, wlo_h, nt, preferred_element_type=jnp.float32)
    addr_ref[...] = jnp.bitwise_or(
        a_low.astype(jnp.int32), lax.shift_left(a_high.astype(jnp.int32), 16))


def _addresses(bits_u8, connections):
    return pl.pallas_call(
        _addr_body,
        grid=(HN // _CBLK,),
        in_specs=[pl.BlockSpec((B, IB), lambda i: (0, 0))] + [
            pl.BlockSpec((64, K), lambda i, s=sub, h=half: (8 * i + 2 * s + h, 0))
            for sub in range(4) for half in range(2)
        ],
        out_specs=pl.BlockSpec((B, _CBLK), lambda i: (0, i)),
        out_shape=jax.ShapeDtypeStruct((B, HN), jnp.int32),
    )(bits_u8, *([connections] * 8))


def _pack_body(mem_ref, packed_ref):
    m = (mem_ref[...] == 1).astype(jnp.bfloat16)
    a = lax.broadcasted_iota(jnp.int32, (NA, 2 * NWORDS), 0)
    w = lax.broadcasted_iota(jnp.int32, (NA, 2 * NWORDS), 1)
    sub = a % PACK
    cond = ((a // PACK) == (w % NWORDS)) & ((sub // 16) == (w // NWORDS))
    pm = jnp.where(cond, 1 << (sub & 15), 0).astype(jnp.bfloat16)
    nn = (((1,), (0,)), ((), ()))
    pk = lax.dot_general(m, pm, nn, preferred_element_type=jnp.float32)
    pki = pk.astype(jnp.int32)
    plo = lax.slice(pki, (0, 0), (_MBLK, NWORDS))
    phi = lax.slice(pki, (0, NWORDS), (_MBLK, 2 * NWORDS))
    packed_ref[...] = jnp.bitwise_or(plo, lax.shift_left(phi, 16))


def _pack_memory(memory):
    return pl.pallas_call(
        _pack_body,
        grid=(N // _MBLK,),
        in_specs=[pl.BlockSpec((_MBLK, NA), lambda i: (i, 0))],
        out_specs=pl.BlockSpec((_MBLK, NWORDS), lambda i: (i, 0)),
        out_shape=jax.ShapeDtypeStruct((N, NWORDS), jnp.int32),
    )(memory)


_NTILES = 32
_NPT = N // _NTILES
_RUN = _NPT // 2
_CB = 256
_LANES = 16


def _lookup_body(addr_hbm, packed_hbm, out_hbm,
                 tab_v, a0, a1, o0, o1, sa0, sa1, so0, so1, stab):
    cid = lax.axis_index("c")
    sid = lax.axis_index("s")
    wid = sid * 2 + cid
    n0 = wid * _NPT
    w0 = wid * _RUN

    lane = lax.iota(jnp.int32, _LANES)
    nvec = [lane + g * _LANES for g in range(_NPT // _LANES)]

    abuf, obuf = (a0, a1), (o0, o1)
    asem, osem = (sa0, sa1), (so0, so1)
    nchunks = B // _CB

    def start_in(c):
        return pltpu.async_copy(
            addr_hbm.at[pl.ds(c * _CB, _CB), pl.ds(w0, _RUN)],
            abuf[c % 2], asem[c % 2])

    def start_out(c):
        return pltpu.async_copy(
            obuf[c % 2],
            out_hbm.at[pl.ds(c * _CB, _CB), pl.ds(n0, _NPT)], osem[c % 2])

    in_cps = [None] * nchunks
    out_cps = [None] * nchunks
    in_cps[0] = start_in(0)
    if nchunks > 1:
        in_cps[1] = start_in(1)
    pltpu.async_copy(packed_hbm.at[pl.ds(n0, _NPT), :], tab_v, stab).wait()
    for c in range(nchunks):
        av, ov = abuf[c % 2], obuf[c % 2]
        if c + 1 < nchunks and in_cps[c + 1] is None:
            in_cps[c + 1] = start_in(c + 1)
        in_cps[c].wait()
        if c >= 2:
            out_cps[c - 2].wait()

        @plsc.parallel_loop(0, _CB, unroll=8)
        def _row(r):
            for gp in range(_RUN // _LANES):
                w = av[r, pl.ds(gp * _LANES, _LANES)]
                for half in range(2):
                    if half == 0:
                        a = jnp.bitwise_and(w, 0xFFF)
                    else:
                        a = lax.shift_right_logical(w, 16)
                    g = gp + 4 * half
                    word = plsc.load_gather(
                        tab_v, [nvec[g], lax.shift_right_logical(a, 5)])
                    bit = jnp.bitwise_and(
                        lax.shift_right_logical(
                            word, jnp.bitwise_and(a, 31)), 1)
                    ov[r, pl.ds(g * _LANES, _LANES)] = bit

        out_cps[c] = start_out(c)
    out_cps[-2].wait()
    out_cps[-1].wait()


def _lookup(addresses, packed):
    mesh = plsc.VectorSubcoreMesh(core_axis_name="c", subcore_axis_name="s")
    f = pl.kernel(
        _lookup_body,
        out_type=jax.ShapeDtypeStruct((B, N), jnp.int32),
        mesh=mesh,
        compiler_params=pltpu.CompilerParams(
            use_tc_tiling_on_sc=False, needs_layout_passes=False
        ),
        scratch_types=[
            pltpu.VMEM((_NPT, NWORDS), jnp.int32),
            pltpu.VMEM((_CB, _RUN), jnp.int32),
            pltpu.VMEM((_CB, _RUN), jnp.int32),
            pltpu.VMEM((_CB, _NPT), jnp.int32),
            pltpu.VMEM((_CB, _NPT), jnp.int32),
            pltpu.SemaphoreType.DMA,
            pltpu.SemaphoreType.DMA,
            pltpu.SemaphoreType.DMA,
            pltpu.SemaphoreType.DMA,
            pltpu.SemaphoreType.DMA,
        ],
    )
    return f(addresses, packed)


def kernel(input_bits, connections, memory):
    bits_u8 = input_bits.astype(jnp.uint8)
    addresses = _addresses(bits_u8, connections)
    packed = _pack_memory(memory)
    out = _lookup(addresses, packed)
    return out.astype(jnp.bool_)


# speedup vs baseline: 1.1300x; 1.0019x over previous
import jax
import jax.numpy as jnp
from jax import lax
from jax.experimental import pallas as pl
from jax.experimental.pallas import tpu as pltpu
from jax.experimental.pallas import tpu_sc as plsc

B = 1024
IB = 1024
N = 4096
HN = N // 2
K = 12
NA = 4096
PACK = 32
NWORDS = NA // PACK


_CBLK = 256
_MBLK = 512


def _wt_quad(conn_refs, ii):
    whi_parts, wlo_parts = [], []
    for cr in conn_refs:
        conn = cr[...]
        whi = jnp.zeros((64, IB), jnp.float32)
        wlo = jnp.zeros((64, IB), jnp.float32)
        for k in range(K // 2):
            w = float(2 ** (K - 1 - k))
            whi = whi + jnp.where(conn[:, k : k + 1] == ii, w, 0.0)
        for k in range(K // 2, K):
            w = float(2 ** (K - 1 - k))
            wlo = wlo + jnp.where(conn[:, k : k + 1] == ii, w, 0.0)
        whi_parts.append(whi)
        wlo_parts.append(wlo)
    return (jnp.concatenate(whi_parts, axis=0).astype(jnp.bfloat16),
            jnp.concatenate(wlo_parts, axis=0).astype(jnp.bfloat16))


def _addr_body(bits_ref, c0, c1, c2, c3, c4, c5, c6, c7, addr_ref):
    bits = bits_ref[...].astype(jnp.bfloat16)
    ii = lax.broadcasted_iota(jnp.int32, (64, IB), 1)
    nt = (((1,), (1,)), ((), ()))

    whi_l, wlo_l = _wt_quad([c0, c2, c4, c6], ii)
    whi_h, wlo_h = _wt_quad([c1, c3, c5, c7], ii)
    a_low = lax.dot_general(
        bits, whi_l, nt, preferred_element_type=jnp.float32)
    a_low = a_low + lax.dot_general(
        bits, wlo_l, nt, preferred_element_type=jnp.float32)
    a_high = lax.dot_general(
        bits, whi_h, nt, preferred_element_type=jnp.float32)
    a_high = a_high + lax.dot_general(
        bits, wlo_h, nt, preferred_element_type=jnp.float32)
    addr_ref[...] = jnp.bitwise_or(
        a_low.astype(jnp.int32), lax.shift_left(a_high.astype(jnp.int32), 16))


def _addresses(bits_u8, connections):
    return pl.pallas_call(
        _addr_body,
        grid=(HN // _CBLK,),
        in_specs=[pl.BlockSpec((B, IB), lambda i: (0, 0))] + [
            pl.BlockSpec((64, K), lambda i, s=sub, h=half: (8 * i + 2 * s + h, 0))
            for sub in range(4) for half in range(2)
        ],
        out_specs=pl.BlockSpec((B, _CBLK), lambda i: (0, i)),
        out_shape=jax.ShapeDtypeStruct((B, HN), jnp.int32),
    )(bits_u8, *([connections] * 8))


def _pack_body(mem_ref, packed_ref):
    m = (mem_ref[...] == 1).astype(jnp.bfloat16)
    a = lax.broadcasted_iota(jnp.int32, (NA, 2 * NWORDS), 0)
    w = lax.broadcasted_iota(jnp.int32, (NA, 2 * NWORDS), 1)
    sub = a % PACK
    cond = ((a // PACK) == (w % NWORDS)) & ((sub // 16) == (w // NWORDS))
    pm = jnp.where(cond, 1 << (sub & 15), 0).astype(jnp.bfloat16)
    nn = (((1,), (0,)), ((), ()))
    pk = lax.dot_general(m, pm, nn, preferred_element_type=jnp.float32)
    pki = pk.astype(jnp.int32)
    plo = lax.slice(pki, (0, 0), (_MBLK, NWORDS))
    phi = lax.slice(pki, (0, NWORDS), (_MBLK, 2 * NWORDS))
    packed_ref[...] = jnp.bitwise_or(plo, lax.shift_left(phi, 16))


def _pack_memory(memory):
    return pl.pallas_call(
        _pack_body,
        grid=(N // _MBLK,),
        in_specs=[pl.BlockSpec((_MBLK, NA), lambda i: (i, 0))],
        out_specs=pl.BlockSpec((_MBLK, NWORDS), lambda i: (i, 0)),
        out_shape=jax.ShapeDtypeStruct((N, NWORDS), jnp.int32),
    )(memory)


_NTILES = 32
_NPT = N // _NTILES
_RUN = _NPT // 2
_CB = 256
_LANES = 16


def _lookup_body(addr_hbm, packed_hbm, out_hbm,
                 tab_v, a0, a1, o0, o1, sa0, sa1, so0, so1, stab):
    cid = lax.axis_index("c")
    sid = lax.axis_index("s")
    wid = sid * 2 + cid
    n0 = wid * _NPT
    w0 = wid * _RUN

    lane = lax.iota(jnp.int32, _LANES)
    nvec = [lane + g * _LANES for g in range(_NPT // _LANES)]

    abuf, obuf = (a0, a1), (o0, o1)
    asem, osem = (sa0, sa1), (so0, so1)
    nchunks = B // _CB

    def start_in(c):
        return pltpu.async_copy(
            addr_hbm.at[pl.ds(c * _CB, _CB), pl.ds(w0, _RUN)],
            abuf[c % 2], asem[c % 2])

    def start_out(c):
        return pltpu.async_copy(
            obuf[c % 2],
            out_hbm.at[pl.ds(c * _CB, _CB), pl.ds(n0, _NPT)], osem[c % 2])

    in_cps = [None] * nchunks
    out_cps = [None] * nchunks
    in_cps[0] = start_in(0)
    if nchunks > 1:
        in_cps[1] = start_in(1)
    pltpu.async_copy(packed_hbm.at[pl.ds(n0, _NPT), :], tab_v, stab).wait()
    for c in range(nchunks):
        av, ov = abuf[c % 2], obuf[c % 2]
        if c + 1 < nchunks and in_cps[c + 1] is None:
            in_cps[c + 1] = start_in(c + 1)
        in_cps[c].wait()
        if c >= 2:
            out_cps[c - 2].wait()

        @plsc.parallel_loop(0, _CB, unroll=8)
        def _row(r):
            for gp in range(_RUN // _LANES):
                w = av[r, pl.ds(gp * _LANES, _LANES)]
                for half in range(2):
                    if half == 0:
                        a = jnp.bitwise_and(w, 0xFFF)
                    else:
                        a = lax.shift_right_logical(w, 16)
                    g = gp + 4 * half
                    word = plsc.load_gather(
                        tab_v, [nvec[g], lax.shift_right_logical(a, 5)])
                    bit = jnp.bitwise_and(
                        lax.shift_right_logical(
                            word, jnp.bitwise_and(a, 31)), 1)
                    ov[r, pl.ds(g * _LANES, _LANES)] = bit

        out_cps[c] = start_out(c)
    out_cps[-2].wait()
    out_cps[-1].wait()


def _lookup(addresses, packed):
    mesh = plsc.VectorSubcoreMesh(core_axis_name="c", subcore_axis_name="s")
    f = pl.kernel(
        _lookup_body,
        out_type=jax.ShapeDtypeStruct((B, N), jnp.int32),
        mesh=mesh,
        compiler_params=pltpu.CompilerParams(
            use_tc_tiling_on_sc=False, needs_layout_passes=False
        ),
        scratch_types=[
            pltpu.VMEM((_NPT, NWORDS), jnp.int32),
            pltpu.VMEM((_CB, _RUN), jnp.int32),
            pltpu.VMEM((_CB, _RUN), jnp.int32),
            pltpu.VMEM((_CB, _NPT), jnp.int32),
            pltpu.VMEM((_CB, _NPT), jnp.int32),
            pltpu.SemaphoreType.DMA,
            pltpu.SemaphoreType.DMA,
            pltpu.SemaphoreType.DMA,
            pltpu.SemaphoreType.DMA,
            pltpu.SemaphoreType.DMA,
        ],
    )
    return f(addresses, packed)


def kernel(input_bits, connections, memory):
    bits_u8 = input_bits.astype(jnp.uint8)
    addresses = _addresses(bits_u8, connections)
    packed = _pack_memory(memory)
    out = _lookup(addresses, packed)
    return out.astype(jnp.bool_)
